# Initial kernel scaffold; baseline (speedup 1.0000x reference)
import functools

import jax
import jax.numpy as jnp
from jax import lax
from jax.experimental import pallas as pl
from jax.experimental.pallas import tpu as pltpu
from jax.experimental.pallas import tpu_sc as plsc

N = 10000
E = 160000
G = 64
HH = 16
EPS = 1e-5

NW = 32
EPW = E // NW
CH = 128
NCH = EPW // CH
TAIL = EPW - NCH * CH
NPAD = 10016
STRIPE = NPAD // 16
DUMMY = N

EB = 2000
NEB = E // EB
NB = 1000
NNB = N // NB


def _node_stats_body(x_ref, s_ref, sq_ref):
    @pl.when(pl.program_id(0) == 0)
    def _():
        s_ref[...] = jnp.zeros_like(s_ref)
        sq_ref[...] = jnp.zeros_like(sq_ref)
    x = x_ref[...]
    s_ref[...] += jnp.sum(x, axis=0, keepdims=True)
    sq_ref[...] += jnp.sum(x * x, axis=0, keepdims=True)


def _node_stats(x_node):
    return pl.pallas_call(
        _node_stats_body,
        grid=(NNB,),
        in_specs=[pl.BlockSpec((NB, 128), lambda i: (i, 0))],
        out_specs=[pl.BlockSpec((1, 128), lambda i: (0, 0)),
                   pl.BlockSpec((1, 128), lambda i: (0, 0))],
        out_shape=[jax.ShapeDtypeStruct((1, 128), jnp.float32),
                   jax.ShapeDtypeStruct((1, 128), jnp.float32)],
    )(x_node)


def _edge_mom_body(x_ref, s1_ref, s2_ref):
    @pl.when(pl.program_id(0) == 0)
    def _():
        s1_ref[...] = jnp.zeros_like(s1_ref)
        s2_ref[...] = jnp.zeros_like(s2_ref)
    x = x_ref[...]
    s1_ref[...] += jnp.sum(x, axis=0, keepdims=True)
    s2_ref[...] += lax.dot_general(x, x, (((0,), (0,)), ((), ())),
                                   preferred_element_type=jnp.float32)


def _edge_moments(x_edge):
    return pl.pallas_call(
        _edge_mom_body,
        grid=(NEB,),
        in_specs=[pl.BlockSpec((EB, HH), lambda i: (i, 0))],
        out_specs=[pl.BlockSpec((1, HH), lambda i: (0, 0)),
                   pl.BlockSpec((HH, HH), lambda i: (0, 0))],
        out_shape=[jax.ShapeDtypeStruct((1, HH), jnp.float32),
                   jax.ShapeDtypeStruct((HH, HH), jnp.float32)],
    )(x_edge)


def _edge_act_body(x_ref, A_ref, c_ref, a_ref, sa_ref, saa_ref):
    @pl.when(pl.program_id(0) == 0)
    def _():
        sa_ref[...] = jnp.zeros_like(sa_ref)
        saa_ref[...] = jnp.zeros_like(saa_ref)
    t = jnp.dot(x_ref[...], A_ref[...],
                preferred_element_type=jnp.float32) + c_ref[...]
    a = jnp.where(t >= 0, t, 0.8 * t)
    a_ref[...] = a
    sa_ref[...] += jnp.sum(a, axis=0, keepdims=True)
    saa_ref[...] += lax.dot_general(a, a, (((0,), (0,)), ((), ())),
                                    preferred_element_type=jnp.float32)


def _edge_activations(x_edge, A, c):
    return pl.pallas_call(
        _edge_act_body,
        grid=(NEB,),
        in_specs=[pl.BlockSpec((EB, HH), lambda i: (i, 0)),
                  pl.BlockSpec((HH, HH), lambda i: (0, 0)),
                  pl.BlockSpec((1, HH), lambda i: (0, 0))],
        out_specs=[pl.BlockSpec((EB, HH), lambda i: (i, 0)),
                   pl.BlockSpec((1, HH), lambda i: (0, 0)),
                   pl.BlockSpec((HH, HH), lambda i: (0, 0))],
        out_shape=[jax.ShapeDtypeStruct((E, HH), jnp.float32),
                   jax.ShapeDtypeStruct((1, HH), jnp.float32),
                   jax.ShapeDtypeStruct((HH, HH), jnp.float32)],
    )(x_edge, A, c)


def _node_emb_body(x_ref, W_ref, c_ref, o_ref):
    o_ref[...] = jnp.dot(x_ref[...], W_ref[...],
                         preferred_element_type=jnp.float32) + c_ref[...]


def _node_embed(x_node, Wn, cn):
    return pl.pallas_call(
        _node_emb_body,
        grid=(NNB,),
        in_specs=[pl.BlockSpec((NB, 128), lambda i: (i, 0)),
                  pl.BlockSpec((128, HH), lambda i: (0, 0)),
                  pl.BlockSpec((1, HH), lambda i: (0, 0))],
        out_specs=[pl.BlockSpec((NB, HH), lambda i: (i, 0))],
        out_shape=[jax.ShapeDtypeStruct((N, HH), jnp.float32)],
    )(x_node, Wn, cn)[0]


def _msg_body(a_ref, hs_ref, W_ref, c_ref, m_ref):
    ew = jnp.dot(a_ref[...], W_ref[...],
                 preferred_element_type=jnp.float32) + c_ref[...]
    hs = hs_ref[...]
    acc = hs[:, 0:1] * ew[:, 0:HH]
    for i in range(1, HH):
        acc += hs[:, i:i + 1] * ew[:, i * HH:(i + 1) * HH]
    m_ref[...] = acc


def _messages(a, hsrc, W2s, c2):
    return pl.pallas_call(
        _msg_body,
        grid=(NEB,),
        in_specs=[pl.BlockSpec((EB, HH), lambda i: (i, 0)),
                  pl.BlockSpec((EB, HH), lambda i: (i, 0)),
                  pl.BlockSpec((HH, HH * HH), lambda i: (0, 0)),
                  pl.BlockSpec((1, HH * HH), lambda i: (0, 0))],
        out_specs=[pl.BlockSpec((EB, HH), lambda i: (i, 0))],
        out_shape=[jax.ShapeDtypeStruct((E, HH), jnp.float32)],
    )(a, hsrc, W2s, c2)[0]


def _gru_math(agg, h, wih_ref, bih_ref, whh_ref, bhh_ref):
    gi = jnp.dot(agg, wih_ref[...],
                 preferred_element_type=jnp.float32) + bih_ref[...]
    gh = jnp.dot(h, whh_ref[...],
                 preferred_element_type=jnp.float32) + bhh_ref[...]
    r = jax.nn.sigmoid(gi[:, 0:HH] + gh[:, 0:HH])
    z = jax.nn.sigmoid(gi[:, HH:2 * HH] + gh[:, HH:2 * HH])
    nn = jnp.tanh(gi[:, 2 * HH:3 * HH] + r * gh[:, 2 * HH:3 * HH])
    return (1.0 - z) * nn + z * h


def _gru0_body(p0_ref, p1_ref, d0_ref, d1_ref, h_ref,
               wih_ref, bih_ref, whh_ref, bhh_ref, hn_ref, deg_ref):
    cnt = d0_ref[...][:N] + d1_ref[...][:N]
    deg = jnp.maximum(cnt, 1.0)
    agg = (p0_ref[...][:N] + p1_ref[...][:N]) / deg
    hn_ref[...] = _gru_math(agg, h_ref[...], wih_ref, bih_ref,
                            whh_ref, bhh_ref)
    deg_ref[...] = deg


def _gru0(p0, p1, d0, d1, h, wih, bih, whh, bhh):
    return pl.pallas_call(
        _gru0_body,
        out_shape=[jax.ShapeDtypeStruct((N, HH), jnp.float32),
                   jax.ShapeDtypeStruct((N, HH), jnp.float32)],
    )(p0, p1, d0, d1, h, wih, bih, whh, bhh)


def _gru1_body(p0_ref, p1_ref, deg_ref, h_ref,
               wih_ref, bih_ref, whh_ref, bhh_ref, hn_ref):
    agg = (p0_ref[...][:N] + p1_ref[...][:N]) / deg_ref[...]
    hn_ref[...] = _gru_math(agg, h_ref[...], wih_ref, bih_ref,
                            whh_ref, bhh_ref)


def _gru1(p0, p1, deg, h, wih, bih, whh, bhh):
    return pl.pallas_call(
        _gru1_body,
        out_shape=[jax.ShapeDtypeStruct((N, HH), jnp.float32)],
    )(p0, p1, deg, h, wih, bih, whh, bhh)[0]


def _set2set_body(h_ref, n2gc_ref, n2gr_ref, wih0_ref, wihr_ref, whh_ref,
                  bih_ref, bhh_ref, bng_ref, bnb_ref, c1w_ref, c1b_ref,
                  c2w_ref, out_ref):
    h = h_ref[...]
    gcol = jax.lax.broadcasted_iota(jnp.int32, (1, G), 1)
    grow = jax.lax.broadcasted_iota(jnp.int32, (G, 1), 0)
    Pb = n2gc_ref[...] == gcol
    Pf = Pb.astype(jnp.float32)
    PTf = (grow == n2gr_ref[...]).astype(jnp.float32)

    q_star = jnp.zeros((G, 2 * HH), jnp.float32)
    hs = [jnp.zeros((G, HH), jnp.float32) for _ in range(4)]
    cs = [jnp.zeros((G, HH), jnp.float32) for _ in range(4)]
    for _ in range(6):
        inp = q_star
        for l in range(4):
            wih = wih0_ref[...] if l == 0 else wihr_ref[l - 1]
            gates = (jnp.dot(inp, wih, preferred_element_type=jnp.float32)
                     + bih_ref[l]
                     + jnp.dot(hs[l], whh_ref[l],
                               preferred_element_type=jnp.float32)
                     + bhh_ref[l])
            gi = gates[:, 0:HH]
            gf = gates[:, HH:2 * HH]
            gg = gates[:, 2 * HH:3 * HH]
            go = gates[:, 3 * HH:4 * HH]
            c = jax.nn.sigmoid(gf) * cs[l] + jax.nn.sigmoid(gi) * jnp.tanh(gg)
            hcur = jax.nn.sigmoid(go) * jnp.tanh(c)
            hs[l] = hcur
            cs[l] = c
            inp = hcur
        q = inp
        qn = jnp.dot(Pf, q, preferred_element_type=jnp.float32)
        e = jnp.sum(h * qn, axis=1, keepdims=True)
        eb = jnp.where(Pb, e, -3e38)
        emax = jnp.max(eb, axis=0, keepdims=True)
        emax_n = jnp.sum(Pf * emax, axis=1, keepdims=True)
        ee = jnp.exp(e - emax_n)
        denom = jnp.sum(Pf * ee, axis=0, keepdims=True)
        denom_n = jnp.sum(Pf * denom, axis=1, keepdims=True)
        alpha = ee / denom_n
        readout = jnp.dot(PTf, alpha * h, preferred_element_type=jnp.float32)
        q_star = jnp.concatenate([q, readout], axis=1)

    m = jnp.mean(q_star, axis=0, keepdims=True)
    v = jnp.mean((q_star - m) ** 2, axis=0, keepdims=True)
    qn_ = (q_star - m) / jnp.sqrt(v + EPS) * bng_ref[...] + bnb_ref[...]
    x1 = jnp.dot(qn_, c1w_ref[...],
                 preferred_element_type=jnp.float32) + c1b_ref[...]
    x1 = jnp.where(x1 >= 0, x1, 0.1 * x1)
    s = jnp.sum(x1 * c2w_ref[...], axis=1, keepdims=True)
    out_ref[...] = jnp.broadcast_to(s, (G, 128))


def _set2set(h, n2gc, n2gr, wih0, wihr, whh, bih, bhh,
             bng, bnb, c1w, c1b, c2w):
    return pl.pallas_call(
        _set2set_body,
        out_shape=[jax.ShapeDtypeStruct((G, 128), jnp.float32)],
    )(h, n2gc, n2gr, wih0, wihr, whh, bih, bhh, bng, bnb, c1w, c1b, c2w)[0]


_SC_MESH = plsc.VectorSubcoreMesh(core_axis_name="c", subcore_axis_name="s")


@functools.partial(
    pl.kernel, mesh=_SC_MESH,
    out_type=jax.ShapeDtypeStruct((E, HH), jnp.float32),
    scratch_types=[pltpu.VMEM((EPW,), jnp.int32),
                   pltpu.VMEM((EPW, HH), jnp.float32),
                   pltpu.SemaphoreType.DMA],
)
def _sc_gather(h_hbm, src_hbm, out_hbm, idx_v, rows_v, sem):
    base = (lax.axis_index("s") * 2 + lax.axis_index("c")) * EPW
    pltpu.sync_copy(src_hbm.at[pl.ds(base, EPW)], idx_v)

    @pl.loop(0, NCH)
    def _fire(j):
        o = j * CH
        pltpu.make_async_copy(h_hbm.at[idx_v.at[pl.ds(o, CH)]],
                              rows_v.at[pl.ds(o, CH)], sem).start()

    pltpu.make_async_copy(h_hbm.at[idx_v.at[pl.ds(NCH * CH, TAIL)]],
                          rows_v.at[pl.ds(NCH * CH, TAIL)], sem).start()

    @pl.loop(0, NCH)
    def _drain(j):
        o = j * CH
        pltpu.make_async_copy(h_hbm.at[idx_v.at[pl.ds(o, CH)]],
                              rows_v.at[pl.ds(o, CH)], sem).wait()

    pltpu.make_async_copy(h_hbm.at[idx_v.at[pl.ds(NCH * CH, TAIL)]],
                          rows_v.at[pl.ds(NCH * CH, TAIL)], sem).wait()
    pltpu.sync_copy(rows_v, out_hbm.at[pl.ds(base, EPW)])


def _sc_scatter_common(msg_hbm, dstp_hbm, zeros_hbm, ones_hbm, shared, sharedd,
                       idx_v, val_v, ones_v, with_deg):
    cid = lax.axis_index("c")
    sid = lax.axis_index("s")
    wid = sid * 2 + cid
    base = wid * EPW
    pltpu.sync_copy(dstp_hbm.at[wid], idx_v)
    if with_deg:
        pltpu.sync_copy(ones_hbm, ones_v)
    zslice = pl.ds(sid * STRIPE, STRIPE)
    pltpu.sync_copy(zeros_hbm.at[zslice], shared.at[zslice])
    if with_deg:
        pltpu.sync_copy(zeros_hbm.at[zslice], sharedd.at[zslice])
    plsc.subcore_barrier()

    @pl.loop(0, NCH)
    def _chunk(j):
        pltpu.sync_copy(msg_hbm.at[pl.ds(base + j * CH, CH)], val_v)
        pltpu.sync_copy(val_v, shared.at[idx_v.at[j]], add=True)
        if with_deg:
            pltpu.sync_copy(ones_v, sharedd.at[idx_v.at[j]], add=True)

    pltpu.sync_copy(msg_hbm.at[pl.ds(base + NCH * CH, TAIL)],
                    val_v.at[pl.ds(0, TAIL)])
    pltpu.sync_copy(val_v, shared.at[idx_v.at[NCH]], add=True)
    if with_deg:
        pltpu.sync_copy(ones_v, sharedd.at[idx_v.at[NCH]], add=True)
    plsc.subcore_barrier()
    return cid, zslice


@functools.partial(
    pl.kernel, mesh=_SC_MESH,
    out_type=[jax.ShapeDtypeStruct((2, NPAD, HH), jnp.float32),
              jax.ShapeDtypeStruct((2, NPAD, HH), jnp.float32)],
    scratch_types=[pltpu.VMEM_SHARED((NPAD, HH), jnp.float32),
                   pltpu.VMEM_SHARED((NPAD, HH), jnp.float32),
                   pltpu.VMEM((NCH + 1, CH), jnp.int32),
                   pltpu.VMEM((CH, HH), jnp.float32),
                   pltpu.VMEM((CH, HH), jnp.float32),
                   pltpu.VMEM((STRIPE, HH), jnp.float32)],
)
def _sc_scatter_deg(msg_hbm, dstp_hbm, zeros_hbm, ones_hbm, out_hbm, deg_hbm,
                    shared, sharedd, idx_v, val_v, ones_v, stripe_v):
    cid, zslice = _sc_scatter_common(msg_hbm, dstp_hbm, zeros_hbm, ones_hbm,
                                     shared, sharedd, idx_v, val_v, ones_v,
                                     True)
    pltpu.sync_copy(shared.at[zslice], stripe_v)
    pltpu.sync_copy(stripe_v, out_hbm.at[cid, zslice])
    pltpu.sync_copy(sharedd.at[zslice], stripe_v)
    pltpu.sync_copy(stripe_v, deg_hbm.at[cid, zslice])


@functools.partial(
    pl.kernel, mesh=_SC_MESH,
    out_type=jax.ShapeDtypeStruct((2, NPAD, HH), jnp.float32),
    scratch_types=[pltpu.VMEM_SHARED((NPAD, HH), jnp.float32),
                   pltpu.VMEM((NCH + 1, CH), jnp.int32),
                   pltpu.VMEM((CH, HH), jnp.float32),
                   pltpu.VMEM((STRIPE, HH), jnp.float32)],
)
def _sc_scatter(msg_hbm, dstp_hbm, zeros_hbm, out_hbm,
                shared, idx_v, val_v, stripe_v):
    cid, zslice = _sc_scatter_common(msg_hbm, dstp_hbm, zeros_hbm, None,
                                     shared, None, idx_v, val_v, None,
                                     False)
    pltpu.sync_copy(shared.at[zslice], stripe_v)
    pltpu.sync_copy(stripe_v, out_hbm.at[cid, zslice])


def kernel(x_node, x_edge, edge_index, node2graph, params):
    p = params
    fE = float(E)
    fN = float(N)

    s_n, sq_n = _node_stats(x_node)
    mu_n = s_n[0] / fN
    var_n = sq_n[0] / fN - mu_n * mu_n
    sn = p['bn_n_g'] / jnp.sqrt(var_n + EPS)
    Wn = sn[:, None] * p['nemb_W'].T
    cn = ((p['bn_n_b'] - mu_n * sn) @ p['nemb_W'].T + p['nemb_b'])[None, :]
    hn = _node_embed(x_node, Wn, cn)

    S1, S2 = _edge_moments(x_edge)
    mu_e = S1[0] / fE
    Cov_e = S2 / fE - mu_e[:, None] * mu_e[None, :]
    var_e = jnp.diagonal(Cov_e)
    se = p['bn_e_g'] / jnp.sqrt(var_e + EPS)
    A0 = se[:, None] * p['eemb_W'].T
    c0 = (p['bn_e_b'] - mu_e * se) @ p['eemb_W'].T + p['eemb_b']
    A1 = A0 @ p['en1_W'].T
    c1 = c0 @ p['en1_W'].T + p['en1_b']
    mean1 = mu_e @ A1 + c1
    var1 = jnp.sum(A1 * (Cov_e @ A1), axis=0)
    s1 = p['enbn1_g'] / jnp.sqrt(var1 + EPS)
    A = A1 * s1[None, :]
    c = ((c1 - mean1) * s1 + p['enbn1_b'])[None, :]

    a, Sa, Saa = _edge_activations(x_edge, A, c)

    mu_a = Sa[0] / fE
    Cov_a = Saa / fE - mu_a[:, None] * mu_a[None, :]
    mean2 = mu_a @ p['en2_W'].T + p['en2_b']
    var2 = jnp.sum(p['en2_W'] * (p['en2_W'] @ Cov_a), axis=1)
    s2 = p['enbn2_g'] / jnp.sqrt(var2 + EPS)
    W2s = p['en2_W'].T * s2[None, :]
    c2 = ((p['en2_b'] - mean2) * s2 + p['enbn2_b'])[None, :]

    src = edge_index[0]
    dst = edge_index[1]
    dstp = jnp.pad(dst.reshape(NW, EPW), ((0, 0), (0, (NCH + 1) * CH - EPW)),
                   constant_values=DUMMY).reshape(NW, NCH + 1, CH)
    zeros_pad = jnp.zeros((NPAD, HH), jnp.float32)
    ones_ch = jnp.ones((CH, HH), jnp.float32)

    wihT = p['gru_Wih'].T
    whhT = p['gru_Whh'].T
    bih = p['gru_bih'][None, :]
    bhh = p['gru_bhh'][None, :]

    hsrc = _sc_gather(hn, src)
    msg = _messages(a, hsrc, W2s, c2)
    part, degp = _sc_scatter_deg(msg, dstp, zeros_pad, ones_ch)
    h, deg = _gru0(part[0], part[1], degp[0], degp[1], hn,
                   wihT, bih, whhT, bhh)

    hsrc = _sc_gather(h, src)
    msg = _messages(a, hsrc, W2s, c2)
    part = _sc_scatter(msg, dstp, zeros_pad)
    h = _gru1(part[0], part[1], deg, h, wihT, bih, whhT, bhh)

    out = _set2set(
        h,
        node2graph[:, None],
        node2graph[None, :],
        p['lstm_Wih0'].T,
        jnp.transpose(p['lstm_Wih_rest'], (0, 2, 1)),
        jnp.transpose(p['lstm_Whh'], (0, 2, 1)),
        p['lstm_bih'][:, None, :],
        p['lstm_bhh'][:, None, :],
        p['bn_o_g'][None, :],
        p['bn_o_b'][None, :],
        p['c1_W'].T,
        p['c1_b'][None, :],
        p['c2_W'],
    )
    return out[:, :1]


# speedup vs baseline: 1.9601x; 1.9601x over previous
import functools

import jax
import jax.numpy as jnp
from jax import lax
from jax.experimental import pallas as pl
from jax.experimental.pallas import tpu as pltpu
from jax.experimental.pallas import tpu_sc as plsc

N = 10000
E = 160000
G = 64
HH = 16
EPS = 1e-5

NW = 32
EPW = E // NW
CH = 128
NCH = EPW // CH
TAIL = EPW - NCH * CH
NPAD = 10016
STRIPE = NPAD // 16
DUMMY = N

EB = 2000
NEB = E // EB
NB = 1000
NNB = N // NB


def _node_stats_body(x_ref, s_ref, sq_ref):
    @pl.when(pl.program_id(0) == 0)
    def _():
        s_ref[...] = jnp.zeros_like(s_ref)
        sq_ref[...] = jnp.zeros_like(sq_ref)
    x = x_ref[...]
    s_ref[...] += jnp.sum(x, axis=0, keepdims=True)
    sq_ref[...] += jnp.sum(x * x, axis=0, keepdims=True)


def _node_stats(x_node):
    return pl.pallas_call(
        _node_stats_body,
        grid=(NNB,),
        in_specs=[pl.BlockSpec((NB, 128), lambda i: (i, 0))],
        out_specs=[pl.BlockSpec((1, 128), lambda i: (0, 0)),
                   pl.BlockSpec((1, 128), lambda i: (0, 0))],
        out_shape=[jax.ShapeDtypeStruct((1, 128), jnp.float32),
                   jax.ShapeDtypeStruct((1, 128), jnp.float32)],
    )(x_node)


def _edge_mom_body(x_ref, s1_ref, s2_ref):
    @pl.when(pl.program_id(0) == 0)
    def _():
        s1_ref[...] = jnp.zeros_like(s1_ref)
        s2_ref[...] = jnp.zeros_like(s2_ref)
    x = x_ref[...]
    s1_ref[...] += jnp.sum(x, axis=0, keepdims=True)
    s2_ref[...] += lax.dot_general(x, x, (((0,), (0,)), ((), ())),
                                   preferred_element_type=jnp.float32)


def _edge_moments(x_edge):
    return pl.pallas_call(
        _edge_mom_body,
        grid=(NEB,),
        in_specs=[pl.BlockSpec((EB, HH), lambda i: (i, 0))],
        out_specs=[pl.BlockSpec((1, HH), lambda i: (0, 0)),
                   pl.BlockSpec((HH, HH), lambda i: (0, 0))],
        out_shape=[jax.ShapeDtypeStruct((1, HH), jnp.float32),
                   jax.ShapeDtypeStruct((HH, HH), jnp.float32)],
    )(x_edge)


def _edge_act_body(x_ref, A_ref, c_ref, a_ref, sa_ref, saa_ref):
    @pl.when(pl.program_id(0) == 0)
    def _():
        sa_ref[...] = jnp.zeros_like(sa_ref)
        saa_ref[...] = jnp.zeros_like(saa_ref)
    t = jnp.dot(x_ref[...], A_ref[...],
                preferred_element_type=jnp.float32) + c_ref[...]
    a = jnp.where(t >= 0, t, 0.8 * t)
    a_ref[...] = a
    sa_ref[...] += jnp.sum(a, axis=0, keepdims=True)
    saa_ref[...] += lax.dot_general(a, a, (((0,), (0,)), ((), ())),
                                    preferred_element_type=jnp.float32)


def _edge_activations(x_edge, A, c):
    return pl.pallas_call(
        _edge_act_body,
        grid=(NEB,),
        in_specs=[pl.BlockSpec((EB, HH), lambda i: (i, 0)),
                  pl.BlockSpec((HH, HH), lambda i: (0, 0)),
                  pl.BlockSpec((1, HH), lambda i: (0, 0))],
        out_specs=[pl.BlockSpec((EB, HH), lambda i: (i, 0)),
                   pl.BlockSpec((1, HH), lambda i: (0, 0)),
                   pl.BlockSpec((HH, HH), lambda i: (0, 0))],
        out_shape=[jax.ShapeDtypeStruct((E, HH), jnp.float32),
                   jax.ShapeDtypeStruct((1, HH), jnp.float32),
                   jax.ShapeDtypeStruct((HH, HH), jnp.float32)],
    )(x_edge, A, c)


def _node_emb_body(x_ref, W_ref, c_ref, o_ref):
    o_ref[...] = jnp.dot(x_ref[...], W_ref[...],
                         preferred_element_type=jnp.float32) + c_ref[...]


def _node_embed(x_node, Wn, cn):
    return pl.pallas_call(
        _node_emb_body,
        grid=(NNB,),
        in_specs=[pl.BlockSpec((NB, 128), lambda i: (i, 0)),
                  pl.BlockSpec((128, HH), lambda i: (0, 0)),
                  pl.BlockSpec((1, HH), lambda i: (0, 0))],
        out_specs=[pl.BlockSpec((NB, HH), lambda i: (i, 0))],
        out_shape=[jax.ShapeDtypeStruct((N, HH), jnp.float32)],
    )(x_node, Wn, cn)[0]


def _msg_body(a_ref, hs_ref, W_ref, c_ref, m_ref):
    ew = jnp.dot(a_ref[...], W_ref[...],
                 preferred_element_type=jnp.float32) + c_ref[...]
    hs = hs_ref[...]
    acc = hs[:, 0:1] * ew[:, 0:HH]
    for i in range(1, HH):
        acc += hs[:, i:i + 1] * ew[:, i * HH:(i + 1) * HH]
    m_ref[...] = acc


def _messages(a, hsrc, W2s, c2):
    return pl.pallas_call(
        _msg_body,
        grid=(NEB,),
        in_specs=[pl.BlockSpec((EB, HH), lambda i: (i, 0)),
                  pl.BlockSpec((EB, HH), lambda i: (i, 0)),
                  pl.BlockSpec((HH, HH * HH), lambda i: (0, 0)),
                  pl.BlockSpec((1, HH * HH), lambda i: (0, 0))],
        out_specs=[pl.BlockSpec((EB, HH), lambda i: (i, 0))],
        out_shape=[jax.ShapeDtypeStruct((E, HH), jnp.float32)],
    )(a, hsrc, W2s, c2)[0]


def _gru_math(agg, h, wih_ref, bih_ref, whh_ref, bhh_ref):
    gi = jnp.dot(agg, wih_ref[...],
                 preferred_element_type=jnp.float32) + bih_ref[...]
    gh = jnp.dot(h, whh_ref[...],
                 preferred_element_type=jnp.float32) + bhh_ref[...]
    r = jax.nn.sigmoid(gi[:, 0:HH] + gh[:, 0:HH])
    z = jax.nn.sigmoid(gi[:, HH:2 * HH] + gh[:, HH:2 * HH])
    nn = jnp.tanh(gi[:, 2 * HH:3 * HH] + r * gh[:, 2 * HH:3 * HH])
    return (1.0 - z) * nn + z * h


def _gru0_body(p0_ref, p1_ref, d0_ref, d1_ref, h_ref,
               wih_ref, bih_ref, whh_ref, bhh_ref, hn_ref, deg_ref):
    cnt = d0_ref[...][:N] + d1_ref[...][:N]
    deg = jnp.maximum(cnt, 1.0)
    agg = (p0_ref[...][:N] + p1_ref[...][:N]) / deg
    hn_ref[...] = _gru_math(agg, h_ref[...], wih_ref, bih_ref,
                            whh_ref, bhh_ref)
    deg_ref[...] = deg


def _gru0(p0, p1, d0, d1, h, wih, bih, whh, bhh):
    return pl.pallas_call(
        _gru0_body,
        out_shape=[jax.ShapeDtypeStruct((N, HH), jnp.float32),
                   jax.ShapeDtypeStruct((N, HH), jnp.float32)],
    )(p0, p1, d0, d1, h, wih, bih, whh, bhh)


def _gru1_body(p0_ref, p1_ref, deg_ref, h_ref,
               wih_ref, bih_ref, whh_ref, bhh_ref, hn_ref):
    agg = (p0_ref[...][:N] + p1_ref[...][:N]) / deg_ref[...]
    hn_ref[...] = _gru_math(agg, h_ref[...], wih_ref, bih_ref,
                            whh_ref, bhh_ref)


def _gru1(p0, p1, deg, h, wih, bih, whh, bhh):
    return pl.pallas_call(
        _gru1_body,
        out_shape=[jax.ShapeDtypeStruct((N, HH), jnp.float32)],
    )(p0, p1, deg, h, wih, bih, whh, bhh)[0]


def _set2set_body(h_ref, n2gc_ref, n2gr_ref, wih0_ref, wihr_ref, whh_ref,
                  bih_ref, bhh_ref, bng_ref, bnb_ref, c1w_ref, c1b_ref,
                  c2w_ref, c2b_ref, out_ref):
    h = h_ref[...]
    gcol = jax.lax.broadcasted_iota(jnp.int32, (1, G), 1)
    grow = jax.lax.broadcasted_iota(jnp.int32, (G, 1), 0)
    Pb = n2gc_ref[...] == gcol
    Pf = Pb.astype(jnp.float32)
    PTf = (grow == n2gr_ref[...]).astype(jnp.float32)

    q_star = jnp.zeros((G, 2 * HH), jnp.float32)
    hs = [jnp.zeros((G, HH), jnp.float32) for _ in range(4)]
    cs = [jnp.zeros((G, HH), jnp.float32) for _ in range(4)]
    for _ in range(6):
        inp = q_star
        for l in range(4):
            wih = wih0_ref[...] if l == 0 else wihr_ref[l - 1]
            gates = (jnp.dot(inp, wih, preferred_element_type=jnp.float32)
                     + bih_ref[l]
                     + jnp.dot(hs[l], whh_ref[l],
                               preferred_element_type=jnp.float32)
                     + bhh_ref[l])
            gi = gates[:, 0:HH]
            gf = gates[:, HH:2 * HH]
            gg = gates[:, 2 * HH:3 * HH]
            go = gates[:, 3 * HH:4 * HH]
            c = jax.nn.sigmoid(gf) * cs[l] + jax.nn.sigmoid(gi) * jnp.tanh(gg)
            hcur = jax.nn.sigmoid(go) * jnp.tanh(c)
            hs[l] = hcur
            cs[l] = c
            inp = hcur
        q = inp
        qn = jnp.dot(Pf, q, preferred_element_type=jnp.float32)
        e = jnp.sum(h * qn, axis=1, keepdims=True)
        eb = jnp.where(Pb, e, -3e38)
        emax = jnp.max(eb, axis=0, keepdims=True)
        emax_n = jnp.sum(Pf * emax, axis=1, keepdims=True)
        ee = jnp.exp(e - emax_n)
        denom = jnp.sum(Pf * ee, axis=0, keepdims=True)
        denom_n = jnp.sum(Pf * denom, axis=1, keepdims=True)
        alpha = ee / denom_n
        readout = jnp.dot(PTf, alpha * h, preferred_element_type=jnp.float32)
        q_star = jnp.concatenate([q, readout], axis=1)

    m = jnp.mean(q_star, axis=0, keepdims=True)
    v = jnp.mean((q_star - m) ** 2, axis=0, keepdims=True)
    qn_ = (q_star - m) / jnp.sqrt(v + EPS) * bng_ref[...] + bnb_ref[...]
    x1 = jnp.dot(qn_, c1w_ref[...],
                 preferred_element_type=jnp.float32) + c1b_ref[...]
    x1 = jnp.where(x1 >= 0, x1, 0.1 * x1)
    s = jnp.sum(x1 * c2w_ref[...], axis=1, keepdims=True) + c2b_ref[...]
    out_ref[...] = jnp.broadcast_to(jax.nn.sigmoid(s), (G, 128))


def _set2set(h, n2gc, n2gr, wih0, wihr, whh, bih, bhh,
             bng, bnb, c1w, c1b, c2w, c2b):
    return pl.pallas_call(
        _set2set_body,
        out_shape=[jax.ShapeDtypeStruct((G, 128), jnp.float32)],
    )(h, n2gc, n2gr, wih0, wihr, whh, bih, bhh, bng, bnb, c1w, c1b, c2w,
      c2b)[0]


def _sc_gather_body(h_hbm, src_hbm, out_hbm, idx_v, rows_v, sem):
    base = (lax.axis_index("s") * 2 + lax.axis_index("c")) * EPW
    pltpu.sync_copy(src_hbm.at[pl.ds(base, EPW)], idx_v)

    @pl.loop(0, NCH)
    def _fire(j):
        o = j * CH
        pltpu.make_async_copy(h_hbm.at[idx_v.at[pl.ds(o, CH)]],
                              rows_v.at[pl.ds(o, CH)], sem).start()

    pltpu.make_async_copy(h_hbm.at[idx_v.at[pl.ds(NCH * CH, TAIL)]],
                          rows_v.at[pl.ds(NCH * CH, TAIL)], sem).start()

    @pl.loop(0, NCH)
    def _drain(j):
        o = j * CH
        pltpu.make_async_copy(h_hbm.at[idx_v.at[pl.ds(o, CH)]],
                              rows_v.at[pl.ds(o, CH)], sem).wait()

    pltpu.make_async_copy(h_hbm.at[idx_v.at[pl.ds(NCH * CH, TAIL)]],
                          rows_v.at[pl.ds(NCH * CH, TAIL)], sem).wait()
    pltpu.sync_copy(rows_v, out_hbm.at[pl.ds(base, EPW)])


def _sc_scatter_common(msg_hbm, dstp_hbm, zeros_hbm, ones_hbm, shared, sharedd,
                       idx_v, val_v, ones_v, with_deg):
    cid = lax.axis_index("c")
    sid = lax.axis_index("s")
    wid = sid * 2 + cid
    base = wid * EPW
    pltpu.sync_copy(dstp_hbm.at[wid], idx_v)
    if with_deg:
        pltpu.sync_copy(ones_hbm, ones_v)
    zslice = pl.ds(sid * STRIPE, STRIPE)
    pltpu.sync_copy(zeros_hbm.at[zslice], shared.at[zslice])
    if with_deg:
        pltpu.sync_copy(zeros_hbm.at[zslice], sharedd.at[zslice])
    plsc.subcore_barrier()

    @pl.loop(0, NCH)
    def _chunk(j):
        pltpu.sync_copy(msg_hbm.at[pl.ds(base + j * CH, CH)], val_v)
        pltpu.sync_copy(val_v, shared.at[idx_v.at[j]], add=True)
        if with_deg:
            pltpu.sync_copy(ones_v, sharedd.at[idx_v.at[j]], add=True)

    pltpu.sync_copy(msg_hbm.at[pl.ds(base + NCH * CH, TAIL)],
                    val_v.at[pl.ds(0, TAIL)])
    pltpu.sync_copy(val_v, shared.at[idx_v.at[NCH]], add=True)
    if with_deg:
        pltpu.sync_copy(ones_v, sharedd.at[idx_v.at[NCH]], add=True)
    plsc.subcore_barrier()
    return cid, zslice


def _sc_scatter_deg_body(msg_hbm, dstp_hbm, zeros_hbm, ones_hbm, out_hbm,
                         deg_hbm, shared, sharedd, idx_v, val_v, ones_v,
                         stripe_v):
    cid, zslice = _sc_scatter_common(msg_hbm, dstp_hbm, zeros_hbm, ones_hbm,
                                     shared, sharedd, idx_v, val_v, ones_v,
                                     True)
    pltpu.sync_copy(shared.at[zslice], stripe_v)
    pltpu.sync_copy(stripe_v, out_hbm.at[cid, zslice])
    pltpu.sync_copy(sharedd.at[zslice], stripe_v)
    pltpu.sync_copy(stripe_v, deg_hbm.at[cid, zslice])


def _sc_scatter_body(msg_hbm, dstp_hbm, zeros_hbm, out_hbm,
                     shared, idx_v, val_v, stripe_v):
    cid, zslice = _sc_scatter_common(msg_hbm, dstp_hbm, zeros_hbm, None,
                                     shared, None, idx_v, val_v, None,
                                     False)
    pltpu.sync_copy(shared.at[zslice], stripe_v)
    pltpu.sync_copy(stripe_v, out_hbm.at[cid, zslice])


@functools.lru_cache(maxsize=1)
def _sc_kernels():
    mesh = plsc.VectorSubcoreMesh(core_axis_name="c", subcore_axis_name="s")
    params = pltpu.CompilerParams(use_tc_tiling_on_sc=False)
    gather = pl.kernel(
        _sc_gather_body, mesh=mesh, compiler_params=params,
        out_type=jax.ShapeDtypeStruct((E, HH), jnp.float32),
        scratch_types=[pltpu.VMEM((EPW,), jnp.int32),
                       pltpu.VMEM((EPW, HH), jnp.float32),
                       pltpu.SemaphoreType.DMA],
    )
    scatter_deg = pl.kernel(
        _sc_scatter_deg_body, mesh=mesh, compiler_params=params,
        out_type=[jax.ShapeDtypeStruct((2, NPAD, HH), jnp.float32),
                  jax.ShapeDtypeStruct((2, NPAD, HH), jnp.float32)],
        scratch_types=[pltpu.VMEM_SHARED((NPAD, HH), jnp.float32),
                       pltpu.VMEM_SHARED((NPAD, HH), jnp.float32),
                       pltpu.VMEM((NCH + 1, CH), jnp.int32),
                       pltpu.VMEM((CH, HH), jnp.float32),
                       pltpu.VMEM((CH, HH), jnp.float32),
                       pltpu.VMEM((STRIPE, HH), jnp.float32)],
    )
    scatter = pl.kernel(
        _sc_scatter_body, mesh=mesh, compiler_params=params,
        out_type=jax.ShapeDtypeStruct((2, NPAD, HH), jnp.float32),
        scratch_types=[pltpu.VMEM_SHARED((NPAD, HH), jnp.float32),
                       pltpu.VMEM((NCH + 1, CH), jnp.int32),
                       pltpu.VMEM((CH, HH), jnp.float32),
                       pltpu.VMEM((STRIPE, HH), jnp.float32)],
    )
    return gather, scatter_deg, scatter


def _sc_gather(h, src):
    return _sc_kernels()[0](h, src)


def _sc_scatter_deg(msg, dstp, zeros_pad, ones_ch):
    return _sc_kernels()[1](msg, dstp, zeros_pad, ones_ch)


def _sc_scatter(msg, dstp, zeros_pad):
    return _sc_kernels()[2](msg, dstp, zeros_pad)


def kernel(x_node, x_edge, edge_index, node2graph, params):
    p = params
    fE = float(E)
    fN = float(N)

    s_n, sq_n = _node_stats(x_node)
    mu_n = s_n[0] / fN
    var_n = sq_n[0] / fN - mu_n * mu_n
    sn = p['bn_n_g'] / jnp.sqrt(var_n + EPS)
    Wn = sn[:, None] * p['nemb_W'].T
    cn = ((p['bn_n_b'] - mu_n * sn) @ p['nemb_W'].T + p['nemb_b'])[None, :]
    hn = _node_embed(x_node, Wn, cn)

    S1, S2 = _edge_moments(x_edge)
    mu_e = S1[0] / fE
    Cov_e = S2 / fE - mu_e[:, None] * mu_e[None, :]
    var_e = jnp.diagonal(Cov_e)
    se = p['bn_e_g'] / jnp.sqrt(var_e + EPS)
    A0 = se[:, None] * p['eemb_W'].T
    c0 = (p['bn_e_b'] - mu_e * se) @ p['eemb_W'].T + p['eemb_b']
    A1 = A0 @ p['en1_W'].T
    c1 = c0 @ p['en1_W'].T + p['en1_b']
    mean1 = mu_e @ A1 + c1
    var1 = jnp.sum(A1 * (Cov_e @ A1), axis=0)
    s1 = p['enbn1_g'] / jnp.sqrt(var1 + EPS)
    A = A1 * s1[None, :]
    c = ((c1 - mean1) * s1 + p['enbn1_b'])[None, :]

    a, Sa, Saa = _edge_activations(x_edge, A, c)

    mu_a = Sa[0] / fE
    Cov_a = Saa / fE - mu_a[:, None] * mu_a[None, :]
    mean2 = mu_a @ p['en2_W'].T + p['en2_b']
    var2 = jnp.sum(p['en2_W'] * (p['en2_W'] @ Cov_a), axis=1)
    s2 = p['enbn2_g'] / jnp.sqrt(var2 + EPS)
    W2s = p['en2_W'].T * s2[None, :]
    c2 = ((p['en2_b'] - mean2) * s2 + p['enbn2_b'])[None, :]

    src = edge_index[0]
    dst = edge_index[1]
    dstp = jnp.pad(dst.reshape(NW, EPW), ((0, 0), (0, (NCH + 1) * CH - EPW)),
                   constant_values=DUMMY).reshape(NW, NCH + 1, CH)
    zeros_pad = jnp.zeros((NPAD, HH), jnp.float32)
    ones_ch = jnp.ones((CH, HH), jnp.float32)

    wihT = p['gru_Wih'].T
    whhT = p['gru_Whh'].T
    bih = p['gru_bih'][None, :]
    bhh = p['gru_bhh'][None, :]

    hsrc = _sc_gather(hn, src)
    msg = _messages(a, hsrc, W2s, c2)
    part, degp = _sc_scatter_deg(msg, dstp, zeros_pad, ones_ch)
    h, deg = _gru0(part[0], part[1], degp[0], degp[1], hn,
                   wihT, bih, whhT, bhh)

    hsrc = _sc_gather(h, src)
    msg = _messages(a, hsrc, W2s, c2)
    part = _sc_scatter(msg, dstp, zeros_pad)
    h = _gru1(part[0], part[1], deg, h, wihT, bih, whhT, bhh)

    out = _set2set(
        h,
        node2graph[:, None],
        node2graph[None, :],
        p['lstm_Wih0'].T,
        jnp.transpose(p['lstm_Wih_rest'], (0, 2, 1)),
        jnp.transpose(p['lstm_Whh'], (0, 2, 1)),
        p['lstm_bih'][:, None, :],
        p['lstm_bhh'][:, None, :],
        p['bn_o_g'][None, :],
        p['bn_o_b'][None, :],
        p['c1_W'].T,
        p['c1_b'][None, :],
        p['c2_W'],
        p['c2_b'][None, :],
    )
    return out[:, :1]
